# baseline (speedup 1.0000x reference)
import functools

import jax
import jax.numpy as jnp
import numpy as np
from jax import lax
from jax.experimental import pallas as pl
from jax.experimental.pallas import tpu as pltpu
from jax.experimental.pallas import tpu_sc as plsc

_B = 32768
_NC = 1
_NS = 16
_NW = _NC * _NS
_CHUNK = _B // _NW
_LANES = 16
_BLK = 128
_NBLK = _CHUNK // _BLK
_GPB = _BLK // _LANES
_SB = 8
_SBTOK = _CHUNK // _SB
_SBGRP = _SBTOK // _LANES


def _rotl32(x, d):
    return ((x << np.uint32(d)) | (x >> np.uint32(32 - d))).astype(np.uint32)


def _threefry2x32_bits(key_hi, key_lo, n):
    x0 = np.zeros(n, dtype=np.uint32)
    x1 = np.arange(n, dtype=np.uint32)
    ks0, ks1 = np.uint32(key_hi), np.uint32(key_lo)
    ks2 = np.uint32(0x1BD11BDA) ^ ks0 ^ ks1
    rot_a, rot_b = (13, 15, 26, 6), (17, 29, 16, 24)

    def rounds(x0, x1, rots):
        for r in rots:
            x0 = (x0 + x1).astype(np.uint32)
            x1 = _rotl32(x1, r) ^ x0
        return x0, x1

    x0 += ks0; x1 += ks1
    x0, x1 = rounds(x0, x1, rot_a); x0 += ks1; x1 += ks2 + np.uint32(1)
    x0, x1 = rounds(x0, x1, rot_b); x0 += ks2; x1 += ks0 + np.uint32(2)
    x0, x1 = rounds(x0, x1, rot_a); x0 += ks0; x1 += ks1 + np.uint32(3)
    x0, x1 = rounds(x0, x1, rot_b); x0 += ks1; x1 += ks2 + np.uint32(4)
    x0, x1 = rounds(x0, x1, rot_a); x0 += ks2; x1 += ks0 + np.uint32(5)
    return x0 ^ x1


_KD_THR = (1832780943, 270669613)
_KD_EXPL = (2350016172, 1168365246)

_THRESHOLDS = (
    (_threefry2x32_bits(*_KD_THR, _B) >> np.uint32(9)) | np.uint32(0x3F800000)
).view(np.float32) - np.float32(1.0)
_EXPLORE = (_threefry2x32_bits(*_KD_EXPL, _B) & np.uint32(1)).astype(np.int32)

_EXPL_WORDS = np.zeros((_B // 32,), dtype=np.uint32)
for _j in range(32):
    _EXPL_WORDS |= _EXPLORE.astype(np.uint32).reshape(-1, 32)[:, _j] << np.uint32(_j)
_PACKED = np.concatenate([_THRESHOLDS, _EXPL_WORDS.view(np.float32)])


@functools.cache
def _build_sc_route():
    mesh = plsc.VectorSubcoreMesh(
        core_axis_name="c", subcore_axis_name="s",
        num_cores=_NC, num_subcores=_NS,
    )

    @functools.partial(
        pl.kernel,
        out_type=(
            jax.ShapeDtypeStruct((_B,), jnp.int32),
            jax.ShapeDtypeStruct((2 * _B,), jnp.int32),
        ),
        mesh=mesh,
        compiler_params=pltpu.CompilerParams(
            needs_layout_passes=False, skip_device_barrier=True
        ),
        scratch_types=[
            pltpu.VMEM((2 * _CHUNK,), jnp.float32),
            pltpu.VMEM((_CHUNK,), jnp.float32),
            pltpu.VMEM((_CHUNK // 32,), jnp.float32),
            pltpu.VMEM((_LANES,), jnp.float32),
            pltpu.VMEM((_CHUNK,), jnp.int32),
            pltpu.VMEM((2 * _CHUNK,), jnp.int32),
            [pltpu.SemaphoreType.DMA] * _SB,
            pltpu.SemaphoreType.DMA,
            pltpu.SemaphoreType.DMA,
        ],
    )
    def sc_route(q_hbm, pk_hbm, m0_hbm, m1_hbm, eps_hbm,
                 act_hbm, srm_hbm,
                 q_v, thr_v, bits_v, eps_v, act_v, srm_v,
                 in_sems, mask_sem, out_sem):
        wid = lax.axis_index("s") * _NC + lax.axis_index("c")
        base = wid * _CHUNK

        in_copies = []
        for sb in range(_SB):
            h = [
                pltpu.async_copy(
                    q_hbm.at[pl.ds(2 * base + 2 * _SBTOK * sb, 2 * _SBTOK)],
                    q_v.at[pl.ds(2 * _SBTOK * sb, 2 * _SBTOK)], in_sems[sb]),
                pltpu.async_copy(
                    pk_hbm.at[pl.ds(base + _SBTOK * sb, _SBTOK)],
                    thr_v.at[pl.ds(_SBTOK * sb, _SBTOK)], in_sems[sb]),
                pltpu.async_copy(
                    pk_hbm.at[pl.ds(
                        pl.multiple_of(
                            _B + (_CHUNK // 32) * wid + (_SBTOK // 32) * sb, 8),
                        _SBTOK // 32)],
                    bits_v.at[pl.ds(_SBTOK * sb // 32, _SBTOK // 32)],
                    in_sems[sb]),
            ]
            if sb == 0:
                h.append(pltpu.async_copy(eps_hbm, eps_v.at[pl.ds(0, 1)],
                                          in_sems[0]))
            in_copies.append(h)
        mask_copies = []
        for b in range(_NBLK):
            mask_copies.append(pltpu.async_copy(
                m0_hbm.at[pl.ds(base + _BLK * b, _BLK)],
                srm_v.at[pl.ds(2 * _BLK * b, _BLK)], mask_sem))
            mask_copies.append(pltpu.async_copy(
                m1_hbm.at[pl.ds(base + _BLK * b, _BLK)],
                srm_v.at[pl.ds(2 * _BLK * b + _BLK, _BLK)], mask_sem))
        lanes = lax.iota(jnp.int32, _LANES)
        shift = (lanes, lanes + 16)

        for sb in range(_SB):
            for c in in_copies[sb]:
                c.wait()
            if sb == 0:
                eps = jnp.full((_LANES,), eps_v[...][0], dtype=jnp.float32)
            _wps = _SBTOK // 32
            _woff = (_wps * sb) // _LANES * _LANES
            wvec = plsc.bitcast(bits_v[pl.ds(_woff, _LANES)], jnp.int32)
            for k in range(_SBGRP):
                tok = _SBTOK * sb + _LANES * k
                qoff = 2 * _SBTOK * sb + 256 * (k // 8) + 16 * (k % 8)
                q0 = q_v[pl.ds(qoff, _LANES)]
                q1 = q_v[pl.ds(qoff + _BLK, _LANES)]
                exploit = (q1 > q0).astype(jnp.int32)
                thr = thr_v[pl.ds(tok, _LANES)]
                word = jnp.full((_LANES,), wvec[_wps * sb - _woff + k // 2],
                                dtype=jnp.int32)
                expl = (word >> shift[k % 2]) & 1
                act = jnp.where(eps > thr, expl, exploit)
                act_v[pl.ds(tok, _LANES)] = act

        act_out = pltpu.async_copy(act_v, act_hbm.at[pl.ds(base, _CHUNK)],
                                   out_sem)
        for c in mask_copies:
            c.wait()
        srm_out = pltpu.async_copy(
            srm_v, srm_hbm.at[pl.ds(2 * base, 2 * _CHUNK)], out_sem)
        srm_out.wait()
        act_out.wait()

    return sc_route


def kernel(q_table_predicted, input_ig_routing_matrix, is_warm_up_period,
           ig_activations, sc_routing_matrix, sc_mask_0, sc_mask_1, eps,
           is_training):
    del input_ig_routing_matrix, is_warm_up_period, ig_activations
    del sc_routing_matrix
    q_lin = (
        q_table_predicted.reshape(_B // _BLK, _BLK, 2)
        .transpose(0, 2, 1)
        .reshape(2 * _B)
    )
    eps_eff = jnp.where(is_training, eps.astype(jnp.float32), jnp.float32(-1.0))
    actions, srm_lin = _build_sc_route()(
        q_lin,
        jnp.asarray(_PACKED),
        sc_mask_0,
        sc_mask_1,
        eps_eff.reshape(1),
    )
    srm = (
        srm_lin.reshape(_B // _BLK, 2, _BLK)
        .transpose(0, 2, 1)
        .reshape(_B, 2)
    )
    return actions, srm


# speedup vs baseline: 1.0205x; 1.0205x over previous
import functools

import jax
import jax.numpy as jnp
import numpy as np
from jax import lax
from jax.experimental import pallas as pl
from jax.experimental.pallas import tpu as pltpu
from jax.experimental.pallas import tpu_sc as plsc

_B = 32768
_NC = 1
_NS = 16
_NW = _NC * _NS
_CHUNK = _B // _NW
_LANES = 16
_BLK = 128
_NBLK = _CHUNK // _BLK
_GPB = _BLK // _LANES
_SB = 4
_SBTOK = _CHUNK // _SB
_SBGRP = _SBTOK // _LANES


def _rotl32(x, d):
    return ((x << np.uint32(d)) | (x >> np.uint32(32 - d))).astype(np.uint32)


def _threefry2x32_bits(key_hi, key_lo, n):
    x0 = np.zeros(n, dtype=np.uint32)
    x1 = np.arange(n, dtype=np.uint32)
    ks0, ks1 = np.uint32(key_hi), np.uint32(key_lo)
    ks2 = np.uint32(0x1BD11BDA) ^ ks0 ^ ks1
    rot_a, rot_b = (13, 15, 26, 6), (17, 29, 16, 24)

    def rounds(x0, x1, rots):
        for r in rots:
            x0 = (x0 + x1).astype(np.uint32)
            x1 = _rotl32(x1, r) ^ x0
        return x0, x1

    x0 += ks0; x1 += ks1
    x0, x1 = rounds(x0, x1, rot_a); x0 += ks1; x1 += ks2 + np.uint32(1)
    x0, x1 = rounds(x0, x1, rot_b); x0 += ks2; x1 += ks0 + np.uint32(2)
    x0, x1 = rounds(x0, x1, rot_a); x0 += ks0; x1 += ks1 + np.uint32(3)
    x0, x1 = rounds(x0, x1, rot_b); x0 += ks1; x1 += ks2 + np.uint32(4)
    x0, x1 = rounds(x0, x1, rot_a); x0 += ks2; x1 += ks0 + np.uint32(5)
    return x0 ^ x1


_KD_THR = (1832780943, 270669613)
_KD_EXPL = (2350016172, 1168365246)

_THRESHOLDS = (
    (_threefry2x32_bits(*_KD_THR, _B) >> np.uint32(9)) | np.uint32(0x3F800000)
).view(np.float32) - np.float32(1.0)
_EXPLORE = (_threefry2x32_bits(*_KD_EXPL, _B) & np.uint32(1)).astype(np.int32)

_EXPL_WORDS = np.zeros((_B // 32,), dtype=np.uint32)
for _j in range(32):
    _EXPL_WORDS |= _EXPLORE.astype(np.uint32).reshape(-1, 32)[:, _j] << np.uint32(_j)
_PACKED = np.concatenate([_THRESHOLDS, _EXPL_WORDS.view(np.float32)])


@functools.cache
def _build_sc_route():
    mesh = plsc.VectorSubcoreMesh(
        core_axis_name="c", subcore_axis_name="s",
        num_cores=_NC, num_subcores=_NS,
    )

    @functools.partial(
        pl.kernel,
        out_type=(
            jax.ShapeDtypeStruct((_B,), jnp.int32),
            jax.ShapeDtypeStruct((2 * _B,), jnp.int32),
        ),
        mesh=mesh,
        compiler_params=pltpu.CompilerParams(
            needs_layout_passes=False, skip_device_barrier=True
        ),
        scratch_types=[
            pltpu.VMEM((2 * _CHUNK,), jnp.float32),
            pltpu.VMEM((_CHUNK,), jnp.float32),
            pltpu.VMEM((_CHUNK // 32,), jnp.float32),
            pltpu.VMEM((_LANES,), jnp.float32),
            pltpu.VMEM((_CHUNK,), jnp.int32),
            pltpu.VMEM((2 * _CHUNK,), jnp.int32),
            [pltpu.SemaphoreType.DMA] * _SB,
            pltpu.SemaphoreType.DMA,
            pltpu.SemaphoreType.DMA,
        ],
    )
    def sc_route(q_hbm, pk_hbm, m0_hbm, m1_hbm, eps_hbm,
                 act_hbm, srm_hbm,
                 q_v, thr_v, bits_v, eps_v, act_v, srm_v,
                 in_sems, mask_sem, out_sem):
        wid = lax.axis_index("s") * _NC + lax.axis_index("c")
        base = wid * _CHUNK

        in_copies = []
        for sb in range(_SB):
            h = [
                pltpu.async_copy(
                    q_hbm.at[pl.ds(2 * base + 2 * _SBTOK * sb, 2 * _SBTOK)],
                    q_v.at[pl.ds(2 * _SBTOK * sb, 2 * _SBTOK)], in_sems[sb]),
                pltpu.async_copy(
                    pk_hbm.at[pl.ds(base + _SBTOK * sb, _SBTOK)],
                    thr_v.at[pl.ds(_SBTOK * sb, _SBTOK)], in_sems[sb]),
                pltpu.async_copy(
                    pk_hbm.at[pl.ds(
                        pl.multiple_of(
                            _B + (_CHUNK // 32) * wid + (_SBTOK // 32) * sb, 8),
                        _SBTOK // 32)],
                    bits_v.at[pl.ds(_SBTOK * sb // 32, _SBTOK // 32)],
                    in_sems[sb]),
            ]
            if sb == 0:
                h.append(pltpu.async_copy(eps_hbm, eps_v.at[pl.ds(0, 1)],
                                          in_sems[0]))
            in_copies.append(h)
        mask_copies = []
        for b in range(_NBLK):
            mask_copies.append(pltpu.async_copy(
                m0_hbm.at[pl.ds(base + _BLK * b, _BLK)],
                srm_v.at[pl.ds(2 * _BLK * b, _BLK)], mask_sem))
            mask_copies.append(pltpu.async_copy(
                m1_hbm.at[pl.ds(base + _BLK * b, _BLK)],
                srm_v.at[pl.ds(2 * _BLK * b + _BLK, _BLK)], mask_sem))
        lanes = lax.iota(jnp.int32, _LANES)
        shift = (lanes, lanes + 16)

        for sb in range(_SB):
            for c in in_copies[sb]:
                c.wait()
            if sb == 0:
                eps = jnp.full((_LANES,), eps_v[...][0], dtype=jnp.float32)
            _wps = _SBTOK // 32
            _woff = (_wps * sb) // _LANES * _LANES
            wvec = plsc.bitcast(bits_v[pl.ds(_woff, _LANES)], jnp.int32)
            for k in range(_SBGRP):
                tok = _SBTOK * sb + _LANES * k
                qoff = 2 * _SBTOK * sb + 256 * (k // 8) + 16 * (k % 8)
                q0 = q_v[pl.ds(qoff, _LANES)]
                q1 = q_v[pl.ds(qoff + _BLK, _LANES)]
                exploit = (q1 > q0).astype(jnp.int32)
                thr = thr_v[pl.ds(tok, _LANES)]
                word = jnp.full((_LANES,), wvec[_wps * sb - _woff + k // 2],
                                dtype=jnp.int32)
                expl = (word >> shift[k % 2]) & 1
                act = jnp.where(eps > thr, expl, exploit)
                act_v[pl.ds(tok, _LANES)] = act

        act_out = pltpu.async_copy(act_v, act_hbm.at[pl.ds(base, _CHUNK)],
                                   out_sem)
        for c in mask_copies:
            c.wait()
        srm_out = pltpu.async_copy(
            srm_v, srm_hbm.at[pl.ds(2 * base, 2 * _CHUNK)], out_sem)
        srm_out.wait()
        act_out.wait()

    return sc_route


def kernel(q_table_predicted, input_ig_routing_matrix, is_warm_up_period,
           ig_activations, sc_routing_matrix, sc_mask_0, sc_mask_1, eps,
           is_training):
    del input_ig_routing_matrix, is_warm_up_period, ig_activations
    del sc_routing_matrix
    q_lin = (
        q_table_predicted.reshape(_B // _BLK, _BLK, 2)
        .transpose(0, 2, 1)
        .reshape(2 * _B)
    )
    eps_eff = jnp.where(is_training, eps.astype(jnp.float32), jnp.float32(-1.0))
    actions, srm_lin = _build_sc_route()(
        q_lin,
        jnp.asarray(_PACKED),
        sc_mask_0,
        sc_mask_1,
        eps_eff.reshape(1),
    )
    srm = (
        srm_lin.reshape(_B // _BLK, 2, _BLK)
        .transpose(0, 2, 1)
        .reshape(_B, 2)
    )
    return actions, srm


# speedup vs baseline: 1.0234x; 1.0028x over previous
import functools

import jax
import jax.numpy as jnp
import numpy as np
from jax import lax
from jax.experimental import pallas as pl
from jax.experimental.pallas import tpu as pltpu
from jax.experimental.pallas import tpu_sc as plsc

_B = 32768
_NC = 1
_NS = 16
_NW = _NC * _NS
_CHUNK = _B // _NW
_LANES = 16
_BLK = 128
_NBLK = _CHUNK // _BLK
_SB = 4
_SBTOK = _CHUNK // _SB
_SBGRP = _SBTOK // _LANES


def _rotl32(x, d):
    return ((x << np.uint32(d)) | (x >> np.uint32(32 - d))).astype(np.uint32)


def _threefry2x32_bits(key_hi, key_lo, n):
    x0 = np.zeros(n, dtype=np.uint32)
    x1 = np.arange(n, dtype=np.uint32)
    ks0, ks1 = np.uint32(key_hi), np.uint32(key_lo)
    ks2 = np.uint32(0x1BD11BDA) ^ ks0 ^ ks1
    rot_a, rot_b = (13, 15, 26, 6), (17, 29, 16, 24)

    def rounds(x0, x1, rots):
        for r in rots:
            x0 = (x0 + x1).astype(np.uint32)
            x1 = _rotl32(x1, r) ^ x0
        return x0, x1

    x0 += ks0; x1 += ks1
    x0, x1 = rounds(x0, x1, rot_a); x0 += ks1; x1 += ks2 + np.uint32(1)
    x0, x1 = rounds(x0, x1, rot_b); x0 += ks2; x1 += ks0 + np.uint32(2)
    x0, x1 = rounds(x0, x1, rot_a); x0 += ks0; x1 += ks1 + np.uint32(3)
    x0, x1 = rounds(x0, x1, rot_b); x0 += ks1; x1 += ks2 + np.uint32(4)
    x0, x1 = rounds(x0, x1, rot_a); x0 += ks2; x1 += ks0 + np.uint32(5)
    return x0 ^ x1


_KD_THR = (1832780943, 270669613)
_KD_EXPL = (2350016172, 1168365246)

_THRESHOLDS = (
    (_threefry2x32_bits(*_KD_THR, _B) >> np.uint32(9)) | np.uint32(0x3F800000)
).view(np.float32) - np.float32(1.0)
_EXPLORE = (_threefry2x32_bits(*_KD_EXPL, _B) & np.uint32(1)).astype(np.int32)

_EXPL_WORDS = np.zeros((_B // 32,), dtype=np.uint32)
for _j in range(32):
    _EXPL_WORDS |= _EXPLORE.astype(np.uint32).reshape(-1, 32)[:, _j] << np.uint32(_j)
_PACKED = np.concatenate([_THRESHOLDS, _EXPL_WORDS.view(np.float32)])


@functools.cache
def _build_sc_route():
    mesh = plsc.VectorSubcoreMesh(
        core_axis_name="c", subcore_axis_name="s",
        num_cores=_NC, num_subcores=_NS,
    )

    @functools.partial(
        pl.kernel,
        out_type=(
            jax.ShapeDtypeStruct((_B,), jnp.int32),
            jax.ShapeDtypeStruct((2 * _B,), jnp.int32),
        ),
        mesh=mesh,
        compiler_params=pltpu.CompilerParams(
            needs_layout_passes=False, skip_device_barrier=True
        ),
        scratch_types=[
            pltpu.VMEM((2 * _CHUNK,), jnp.float32),
            pltpu.VMEM((_CHUNK,), jnp.float32),
            pltpu.VMEM((_CHUNK // 32,), jnp.float32),
            pltpu.VMEM((_LANES,), jnp.float32),
            pltpu.VMEM((_CHUNK,), jnp.int32),
            pltpu.VMEM((2 * _CHUNK,), jnp.int32),
            [pltpu.SemaphoreType.DMA] * _SB,
            pltpu.SemaphoreType.DMA,
            pltpu.SemaphoreType.DMA,
        ],
    )
    def sc_route(q_hbm, pk_hbm, m0_hbm, m1_hbm, eps_hbm,
                 act_hbm, srm_hbm,
                 q_v, thr_v, bits_v, eps_v, act_v, srm_v,
                 in_sems, mask_sem, out_sem):
        wid = lax.axis_index("s") * _NC + lax.axis_index("c")
        base = wid * _CHUNK

        in_copies = []
        for sb in range(_SB):
            h = [
                pltpu.async_copy(
                    q_hbm.at[pl.ds(2 * base + 2 * _SBTOK * sb, 2 * _SBTOK)],
                    q_v.at[pl.ds(2 * _SBTOK * sb, 2 * _SBTOK)], in_sems[sb]),
                pltpu.async_copy(
                    pk_hbm.at[pl.ds(base + _SBTOK * sb, _SBTOK)],
                    thr_v.at[pl.ds(_SBTOK * sb, _SBTOK)], in_sems[sb]),
                pltpu.async_copy(
                    pk_hbm.at[pl.ds(
                        pl.multiple_of(
                            _B + (_CHUNK // 32) * wid + (_SBTOK // 32) * sb, 8),
                        _SBTOK // 32)],
                    bits_v.at[pl.ds(_SBTOK * sb // 32, _SBTOK // 32)],
                    in_sems[sb]),
            ]
            if sb == 0:
                h.append(pltpu.async_copy(eps_hbm, eps_v.at[pl.ds(0, 1)],
                                          in_sems[0]))
            in_copies.append(h)
        mask_copies = []
        for b in range(_NBLK):
            mask_copies.append(pltpu.async_copy(
                m0_hbm.at[pl.ds(base + _BLK * b, _BLK)],
                srm_v.at[pl.ds(2 * _BLK * b, _BLK)], mask_sem))
            mask_copies.append(pltpu.async_copy(
                m1_hbm.at[pl.ds(base + _BLK * b, _BLK)],
                srm_v.at[pl.ds(2 * _BLK * b + _BLK, _BLK)], mask_sem))
        lanes = lax.iota(jnp.int32, _LANES)
        shift = (lanes, lanes + 16)

        for sb in range(_SB):
            for c in in_copies[sb]:
                c.wait()
            if sb == 0:
                eps = jnp.full((_LANES,), eps_v[...][0], dtype=jnp.float32)
            _wps = _SBTOK // 32
            _woff = (_wps * sb) // _LANES * _LANES
            wvec = plsc.bitcast(bits_v[pl.ds(_woff, _LANES)], jnp.int32)
            for k in range(_SBGRP):
                tok = _SBTOK * sb + _LANES * k
                qoff = 2 * _SBTOK * sb + 256 * (k // 8) + 16 * (k % 8)
                q0 = q_v[pl.ds(qoff, _LANES)]
                q1 = q_v[pl.ds(qoff + _BLK, _LANES)]
                exploit = (q1 > q0).astype(jnp.int32)
                thr = thr_v[pl.ds(tok, _LANES)]
                word = jnp.full((_LANES,), wvec[_wps * sb - _woff + k // 2],
                                dtype=jnp.int32)
                expl = (word >> shift[k % 2]) & 1
                act = jnp.where(eps > thr, expl, exploit)
                act_v[pl.ds(tok, _LANES)] = act

        act_out = pltpu.async_copy(act_v, act_hbm.at[pl.ds(base, _CHUNK)],
                                   out_sem)
        for c in mask_copies:
            c.wait()
        srm_out = pltpu.async_copy(
            srm_v, srm_hbm.at[pl.ds(2 * base, 2 * _CHUNK)], out_sem)
        srm_out.wait()
        act_out.wait()

    return sc_route


def kernel(q_table_predicted, input_ig_routing_matrix, is_warm_up_period,
           ig_activations, sc_routing_matrix, sc_mask_0, sc_mask_1, eps,
           is_training):
    del input_ig_routing_matrix, is_warm_up_period, ig_activations
    del sc_routing_matrix
    q_lin = (
        q_table_predicted.reshape(_B // _BLK, _BLK, 2)
        .transpose(0, 2, 1)
        .reshape(2 * _B)
    )
    eps_eff = jnp.where(is_training, eps.astype(jnp.float32), jnp.float32(-1.0))
    actions, srm_lin = _build_sc_route()(
        q_lin,
        jnp.asarray(_PACKED),
        sc_mask_0,
        sc_mask_1,
        eps_eff.reshape(1),
    )
    srm = (
        srm_lin.reshape(_B // _BLK, 2, _BLK)
        .transpose(0, 2, 1)
        .reshape(_B, 2)
    )
    return actions, srm
